# baseline (speedup 1.0000x reference)
import functools

import jax
import jax.numpy as jnp
from jax import lax
from jax.experimental import pallas as pl
from jax.experimental.pallas import tpu as pltpu
from jax.experimental.pallas import tpu_sc as plsc

_B = 16384
_F = 26
_V = 100000
_D = 32
_NUM = 13

_VQ = 25600
_QW = 25600

_NC = 2
_NS = 16
_NW = _NC * _NS

_ROWS = _B * _F
_RPW = _ROWS // _NW
_CH = 128
_NCH = _RPW // _CH
_GS = 8
_NG = _NCH // _GS

_BLK = 2048


def _tp_body(t0, t1, t2, t3, tout):
    x = jnp.concatenate([t0[0], t1[0], t2[0], t3[0]], axis=0)
    tout[...] = x.T


def _transpose_pack(tt):
    nvc = _VQ // _QW
    in_specs = [
        pl.BlockSpec(
            (1, _D, _QW),
            functools.partial(lambda q, f, vc: (f, 0, (_VQ // _QW) * q + vc), q),
        )
        for q in range(4)
    ]
    return pl.pallas_call(
        _tp_body,
        grid=(_F, nvc),
        in_specs=in_specs,
        out_specs=pl.BlockSpec((_QW, 128), lambda f, vc: (nvc * f + vc, 0)),
        out_shape=jax.ShapeDtypeStruct((_F * _VQ, 128), jnp.float32),
    )(tt, tt, tt, tt)


def _gather_body(ng, gs, table_hbm, idx_hbm, out_hbm, idx_v, buf0, buf1, gsem, wsem):
    wid = lax.axis_index("s") * _NC + lax.axis_index("c")
    pltpu.sync_copy(idx_hbm.at[wid], idx_v)
    bufs = (buf0, buf1)
    pending = [None, None]
    for g in range(ng):
        buf = bufs[g % 2]
        if pending[g % 2] is not None:
            pending[g % 2].wait()
        cps = [
            pltpu.async_copy(table_hbm.at[idx_v.at[g * gs + j]], buf.at[j], gsem)
            for j in range(gs)
        ]
        for cp in cps:
            cp.wait()
        pending[g % 2] = pltpu.async_copy(buf, out_hbm.at[wid, g], wsem)
    for wcp in pending:
        if wcp is not None:
            wcp.wait()


def _make_gather(nch, gs):
    ng = nch // gs
    return functools.partial(
        pl.kernel,
        mesh=plsc.VectorSubcoreMesh(core_axis_name="c", subcore_axis_name="s"),
        out_type=jax.ShapeDtypeStruct((_NW, ng, gs, _CH, _D), jnp.float32),
        scratch_types=[
            pltpu.VMEM((nch, _CH), jnp.int32),
            pltpu.VMEM((gs, _CH, _D), jnp.float32),
            pltpu.VMEM((gs, _CH, _D), jnp.float32),
            pltpu.SemaphoreType.DMA,
            pltpu.SemaphoreType.DMA,
        ],
        compiler_params=pltpu.CompilerParams(use_tc_tiling_on_sc=False),
    )(functools.partial(_gather_body, ng, gs))


_gather_half = _make_gather(_NCH // 2, _GS // 2)


def _mlp_body(emb_ref, num_ref, w1a, w1b, b1, w2, b2, w3, b3, w4, b4, out_ref):
    x = emb_ref[...]
    h = jnp.dot(x, w1a[...], preferred_element_type=jnp.float32)
    h += jnp.dot(num_ref[...], w1b[...], preferred_element_type=jnp.float32)
    h = jnp.maximum(h + b1[...], 0.0)
    h = jnp.maximum(jnp.dot(h, w2[...], preferred_element_type=jnp.float32) + b2[...], 0.0)
    h = jnp.maximum(jnp.dot(h, w3[...], preferred_element_type=jnp.float32) + b3[...], 0.0)
    out_ref[...] = jnp.dot(h, w4[...], preferred_element_type=jnp.float32) + b4[...]


def _full(shape):
    return pl.BlockSpec(shape, lambda i: (0, 0))


def _mlp(emb, num, w1a, w1b, b1, w2, b2, w3, b3, w4, b4):
    rows = emb.shape[0]
    grid = (rows // _BLK,)
    return pl.pallas_call(
        _mlp_body,
        grid=grid,
        in_specs=[
            pl.BlockSpec((_BLK, _F * _D), lambda i: (i, 0)),
            pl.BlockSpec((_BLK, _NUM), lambda i: (i, 0)),
            _full(w1a.shape),
            _full(w1b.shape),
            _full(b1.shape),
            _full(w2.shape),
            _full(b2.shape),
            _full(w3.shape),
            _full(b3.shape),
            _full(w4.shape),
            _full(b4.shape),
        ],
        out_specs=pl.BlockSpec((_BLK, 1), lambda i: (i, 0)),
        out_shape=jax.ShapeDtypeStruct((rows, 1), jnp.float32),
    )(emb, num, w1a, w1b, b1, w2, b2, w3, b3, w4, b4)


def kernel(categorical_inputs, numeric_inputs, tables, W1, b1, W2, b2, W3, b3, W4, b4):
    tt = tables.transpose(0, 2, 1)
    tw = _transpose_pack(tt)
    table_rows = tw.reshape(_F * _VQ * 4, _D)

    v = categorical_inputs
    q = v // _VQ
    r = v - q * _VQ
    offs = (jnp.arange(_F, dtype=jnp.int32) * (4 * _VQ))[None, :]
    fine = offs + 4 * r + q

    hb = _B // 2
    weights = (
        W1[: _F * _D],
        W1[_F * _D :],
        b1.reshape(1, -1),
        W2,
        b2.reshape(1, -1),
        W3,
        b3.reshape(1, -1),
        W4,
        b4.reshape(1, -1),
    )
    outs = []
    for h in range(2):
        idx_h = fine[h * hb : (h + 1) * hb].reshape(_NW, _NCH // 2, _CH)
        emb_h = _gather_half(table_rows, idx_h).reshape(hb, _F * _D)
        num_h = numeric_inputs[h * hb : (h + 1) * hb]
        outs.append(_mlp(emb_h, num_h, *weights))
    return jnp.concatenate(outs, axis=0).reshape(_B)


# speedup vs baseline: 1.0028x; 1.0028x over previous
import functools

import jax
import jax.numpy as jnp
from jax import lax
from jax.experimental import pallas as pl
from jax.experimental.pallas import tpu as pltpu
from jax.experimental.pallas import tpu_sc as plsc

_B = 16384
_F = 26
_V = 100000
_D = 32
_NUM = 13

_VQ = 25600
_QW = 25600

_NC = 2
_NS = 16
_NW = _NC * _NS

_ROWS = _B * _F
_RPW = _ROWS // _NW
_CH = 128
_NCH = _RPW // _CH
_GS = 8
_NG = _NCH // _GS

_BLK = 2048


def _tp_body(t0, t1, t2, t3, tout):
    x = jnp.concatenate([t0[0], t1[0], t2[0], t3[0]], axis=0)
    tout[...] = x.T


def _transpose_pack(tt):
    nvc = _VQ // _QW
    in_specs = [
        pl.BlockSpec(
            (1, _D, _QW),
            functools.partial(lambda q, f, vc: (f, 0, (_VQ // _QW) * q + vc), q),
        )
        for q in range(4)
    ]
    return pl.pallas_call(
        _tp_body,
        grid=(_F, nvc),
        in_specs=in_specs,
        out_specs=pl.BlockSpec((_QW, 128), lambda f, vc: (nvc * f + vc, 0)),
        out_shape=jax.ShapeDtypeStruct((_F * _VQ, 128), jnp.float32),
    )(tt, tt, tt, tt)


def _gather_body(ng, gs, table_hbm, idx_hbm, out_hbm, idx_v, buf0, buf1, gsem, wsem):
    wid = lax.axis_index("s") * _NC + lax.axis_index("c")
    pltpu.sync_copy(idx_hbm.at[wid], idx_v)
    bufs = (buf0, buf1)
    pending = [None, None]
    for g in range(ng):
        buf = bufs[g % 2]
        if pending[g % 2] is not None:
            pending[g % 2].wait()
        cps = [
            pltpu.async_copy(table_hbm.at[idx_v.at[g * gs + j]], buf.at[j], gsem)
            for j in range(gs)
        ]
        for cp in cps:
            cp.wait()
        pending[g % 2] = pltpu.async_copy(buf, out_hbm.at[wid, g], wsem)
    for wcp in pending:
        if wcp is not None:
            wcp.wait()


def _make_gather(nch, gs):
    ng = nch // gs
    return functools.partial(
        pl.kernel,
        mesh=plsc.VectorSubcoreMesh(core_axis_name="c", subcore_axis_name="s"),
        out_type=jax.ShapeDtypeStruct((_NW, ng, gs, _CH, _D), jnp.float32),
        scratch_types=[
            pltpu.VMEM((nch, _CH), jnp.int32),
            pltpu.VMEM((gs, _CH, _D), jnp.float32),
            pltpu.VMEM((gs, _CH, _D), jnp.float32),
            pltpu.SemaphoreType.DMA,
            pltpu.SemaphoreType.DMA,
        ],
        compiler_params=pltpu.CompilerParams(use_tc_tiling_on_sc=False),
    )(functools.partial(_gather_body, ng, gs))


_gather_half = _make_gather(_NCH // 2, 13)


def _mlp_body(emb_ref, num_ref, w1a, w1b, b1, w2, b2, w3, b3, w4, b4, out_ref):
    x = emb_ref[...]
    h = jnp.dot(x, w1a[...], preferred_element_type=jnp.float32)
    h += jnp.dot(num_ref[...], w1b[...], preferred_element_type=jnp.float32)
    h = jnp.maximum(h + b1[...], 0.0)
    h = jnp.maximum(jnp.dot(h, w2[...], preferred_element_type=jnp.float32) + b2[...], 0.0)
    h = jnp.maximum(jnp.dot(h, w3[...], preferred_element_type=jnp.float32) + b3[...], 0.0)
    out_ref[...] = jnp.dot(h, w4[...], preferred_element_type=jnp.float32) + b4[...]


def _full(shape):
    return pl.BlockSpec(shape, lambda i: (0, 0))


def _mlp(emb, num, w1a, w1b, b1, w2, b2, w3, b3, w4, b4):
    rows = emb.shape[0]
    grid = (rows // _BLK,)
    return pl.pallas_call(
        _mlp_body,
        grid=grid,
        in_specs=[
            pl.BlockSpec((_BLK, _F * _D), lambda i: (i, 0)),
            pl.BlockSpec((_BLK, _NUM), lambda i: (i, 0)),
            _full(w1a.shape),
            _full(w1b.shape),
            _full(b1.shape),
            _full(w2.shape),
            _full(b2.shape),
            _full(w3.shape),
            _full(b3.shape),
            _full(w4.shape),
            _full(b4.shape),
        ],
        out_specs=pl.BlockSpec((_BLK, 1), lambda i: (i, 0)),
        out_shape=jax.ShapeDtypeStruct((rows, 1), jnp.float32),
    )(emb, num, w1a, w1b, b1, w2, b2, w3, b3, w4, b4)


def kernel(categorical_inputs, numeric_inputs, tables, W1, b1, W2, b2, W3, b3, W4, b4):
    tt = tables.transpose(0, 2, 1)
    tw = _transpose_pack(tt)
    table_rows = tw.reshape(_F * _VQ * 4, _D)

    v = categorical_inputs
    q = v // _VQ
    r = v - q * _VQ
    offs = (jnp.arange(_F, dtype=jnp.int32) * (4 * _VQ))[None, :]
    fine = offs + 4 * r + q

    hb = _B // 2
    weights = (
        W1[: _F * _D],
        W1[_F * _D :],
        b1.reshape(1, -1),
        W2,
        b2.reshape(1, -1),
        W3,
        b3.reshape(1, -1),
        W4,
        b4.reshape(1, -1),
    )
    outs = []
    for h in range(2):
        idx_h = fine[h * hb : (h + 1) * hb].reshape(_NW, _NCH // 2, _CH)
        emb_h = _gather_half(table_rows, idx_h).reshape(hb, _F * _D)
        num_h = numeric_inputs[h * hb : (h + 1) * hb]
        outs.append(_mlp(emb_h, num_h, *weights))
    return jnp.concatenate(outs, axis=0).reshape(_B)
